# Initial kernel scaffold; baseline (speedup 1.0000x reference)
import functools

import jax
import jax.numpy as jnp
from jax import lax
from jax.experimental import pallas as pl
from jax.experimental.pallas import tpu as pltpu
from jax.experimental.pallas import tpu_sc as plsc

VOCAB = 100000
DIM = 128
BATCH = 16384

NC, NS = 2, 16
NW = NC * NS
BPW = BATCH // NW
CHUNK = 128
NCHUNK = BPW // CHUNK
NLANE = 16
NVEC = DIM // NLANE


def _body(w_in_hbm, w_out_hbm, idx_in_hbm, idx_out_hbm, out_hbm,
          idx_in_v, idx_out_v, rows_in_v, rows_out_v, out_v,
          sem_in, sem_out):
    wid = lax.axis_index("s") * NC + lax.axis_index("c")
    base = wid * BPW
    pltpu.sync_copy(idx_in_hbm.at[pl.ds(base, BPW)], idx_in_v)
    pltpu.sync_copy(idx_out_hbm.at[pl.ds(base, BPW)], idx_out_v)

    for c in range(NCHUNK):
        pltpu.async_copy(
            w_in_hbm.at[idx_in_v.at[pl.ds(c * CHUNK, CHUNK)]],
            rows_in_v, sem_in).wait()
        pltpu.async_copy(
            w_out_hbm.at[idx_out_v.at[pl.ds(c * CHUNK, CHUNK)]],
            rows_out_v, sem_out).wait()

        def pair(p, carry, c=c):
            acc = (rows_in_v[p, pl.ds(0, NLANE)]
                   * rows_out_v[p, pl.ds(0, NLANE)])
            for j in range(1, NVEC):
                acc = acc + (rows_in_v[p, pl.ds(j * NLANE, NLANE)]
                             * rows_out_v[p, pl.ds(j * NLANE, NLANE)])
            out_v[c * CHUNK + p] = jnp.sum(acc)
            return carry

        lax.fori_loop(0, CHUNK, pair, 0)

    pltpu.sync_copy(out_v, out_hbm.at[pl.ds(base, BPW)])


@functools.partial(
    pl.kernel,
    out_type=jax.ShapeDtypeStruct((BATCH,), jnp.float32),
    mesh=plsc.VectorSubcoreMesh(core_axis_name="c", subcore_axis_name="s"),
    scratch_types=[
        pltpu.VMEM((BPW,), jnp.int32),
        pltpu.VMEM((BPW,), jnp.int32),
        pltpu.VMEM((CHUNK, DIM), jnp.float32),
        pltpu.VMEM((CHUNK, DIM), jnp.float32),
        pltpu.VMEM((BPW,), jnp.float32),
        pltpu.SemaphoreType.DMA,
        pltpu.SemaphoreType.DMA,
    ],
)
def _sc_dot(w_in_hbm, w_out_hbm, idx_in_hbm, idx_out_hbm, out_hbm,
            idx_in_v, idx_out_v, rows_in_v, rows_out_v, out_v,
            sem_in, sem_out):
    _body(w_in_hbm, w_out_hbm, idx_in_hbm, idx_out_hbm, out_hbm,
          idx_in_v, idx_out_v, rows_in_v, rows_out_v, out_v,
          sem_in, sem_out)


def kernel(x, W_in, W_out):
    idx_in = x[:, 0].astype(jnp.int32)
    idx_out = x[:, 1].astype(jnp.int32)
    return _sc_dot(W_in, W_out, idx_in, idx_out)


# speedup vs baseline: 1.0508x; 1.0508x over previous
import functools

import jax
import jax.numpy as jnp
from jax import lax
from jax.experimental import pallas as pl
from jax.experimental.pallas import tpu as pltpu
from jax.experimental.pallas import tpu_sc as plsc

VOCAB = 100000
DIM = 128
BATCH = 16384

NC, NS = 2, 16
NW = NC * NS
BPW = BATCH // NW
CHUNK = 128
NCHUNK = BPW // CHUNK
NLANE = 16
NVEC = DIM // NLANE


def _body(w_in_hbm, w_out_hbm, idx_in_hbm, idx_out_hbm, out_hbm,
          idx_in_v, idx_out_v, rows_in_v, rows_out_v, prods_v, out_v,
          sem_in, sem_out):
    wid = lax.axis_index("s") * NC + lax.axis_index("c")
    base = wid * BPW
    pltpu.sync_copy(idx_in_hbm.at[pl.ds(base, BPW)], idx_in_v)
    pltpu.sync_copy(idx_out_hbm.at[pl.ds(base, BPW)], idx_out_v)

    for c in range(NCHUNK):
        pltpu.async_copy(
            w_in_hbm.at[idx_in_v.at[pl.ds(c * CHUNK, CHUNK)]],
            rows_in_v, sem_in).wait()
        pltpu.async_copy(
            w_out_hbm.at[idx_out_v.at[pl.ds(c * CHUNK, CHUNK)]],
            rows_out_v, sem_out).wait()

        lane = lax.iota(jnp.int32, NLANE)

        def group(g, carry, c=c, lane=lane):
            base_p = g * NLANE
            for p in range(NLANE):
                pair = base_p + p
                acc = (rows_in_v[pair, pl.ds(0, NLANE)]
                       * rows_out_v[pair, pl.ds(0, NLANE)])
                for j in range(1, NVEC):
                    acc = acc + (rows_in_v[pair, pl.ds(j * NLANE, NLANE)]
                                 * rows_out_v[pair, pl.ds(j * NLANE, NLANE)])
                prods_v[p, :] = acc
            dot = plsc.load_gather(
                prods_v, [lane, jnp.zeros((NLANE,), jnp.int32)])
            for j in range(1, NLANE):
                dot = dot + plsc.load_gather(
                    prods_v, [lane, jnp.full((NLANE,), j, jnp.int32)])
            out_v[pl.ds(c * CHUNK + base_p, NLANE)] = dot
            return carry

        lax.fori_loop(0, CHUNK // NLANE, group, 0)

    pltpu.sync_copy(out_v, out_hbm.at[pl.ds(base, BPW)])


@functools.partial(
    pl.kernel,
    out_type=jax.ShapeDtypeStruct((BATCH,), jnp.float32),
    mesh=plsc.VectorSubcoreMesh(core_axis_name="c", subcore_axis_name="s"),
    compiler_params=pltpu.CompilerParams(needs_layout_passes=False),
    scratch_types=[
        pltpu.VMEM((BPW,), jnp.int32),
        pltpu.VMEM((BPW,), jnp.int32),
        pltpu.VMEM((CHUNK, DIM), jnp.float32),
        pltpu.VMEM((CHUNK, DIM), jnp.float32),
        pltpu.VMEM((NLANE, NLANE), jnp.float32),
        pltpu.VMEM((BPW,), jnp.float32),
        pltpu.SemaphoreType.DMA,
        pltpu.SemaphoreType.DMA,
    ],
)
def _sc_dot(w_in_hbm, w_out_hbm, idx_in_hbm, idx_out_hbm, out_hbm,
            idx_in_v, idx_out_v, rows_in_v, rows_out_v, prods_v, out_v,
            sem_in, sem_out):
    _body(w_in_hbm, w_out_hbm, idx_in_hbm, idx_out_hbm, out_hbm,
          idx_in_v, idx_out_v, rows_in_v, rows_out_v, prods_v, out_v,
          sem_in, sem_out)


def kernel(x, W_in, W_out):
    idx_in = x[:, 0].astype(jnp.int32)
    idx_out = x[:, 1].astype(jnp.int32)
    return _sc_dot(W_in, W_out, idx_in, idx_out)


# speedup vs baseline: 1.2559x; 1.1952x over previous
import functools

import jax
import jax.numpy as jnp
from jax import lax
from jax.experimental import pallas as pl
from jax.experimental.pallas import tpu as pltpu
from jax.experimental.pallas import tpu_sc as plsc

VOCAB = 100000
DIM = 128
BATCH = 16384

NC, NS = 2, 16
NW = NC * NS
BPW = BATCH // NW
CHUNK = 128
NCHUNK = BPW // CHUNK
NLANE = 16
NVEC = DIM // NLANE


def _body(w_in_hbm, w_out_hbm, idx_in_hbm, idx_out_hbm, out_hbm,
          idx_in_v, idx_out_v, rows_in_0, rows_out_0, rows_in_1, rows_out_1,
          prods_v, out_v, sem_0, sem_1):
    wid = lax.axis_index("s") * NC + lax.axis_index("c")
    base = wid * BPW
    pltpu.sync_copy(idx_in_hbm.at[pl.ds(base, BPW)], idx_in_v)
    pltpu.sync_copy(idx_out_hbm.at[pl.ds(base, BPW)], idx_out_v)

    rows_in = (rows_in_0, rows_in_1)
    rows_out = (rows_out_0, rows_out_1)
    sems = (sem_0, sem_1)

    def issue(c):
        s = c % 2
        d1 = pltpu.async_copy(
            w_in_hbm.at[idx_in_v.at[pl.ds(c * CHUNK, CHUNK)]],
            rows_in[s], sems[s])
        d2 = pltpu.async_copy(
            w_out_hbm.at[idx_out_v.at[pl.ds(c * CHUNK, CHUNK)]],
            rows_out[s], sems[s])
        return d1, d2

    descs = issue(0)
    for c in range(NCHUNK):
        nxt = issue(c + 1) if c + 1 < NCHUNK else None
        descs[0].wait()
        descs[1].wait()
        descs = nxt
        rows_in_v = rows_in[c % 2]
        rows_out_v = rows_out[c % 2]

        lane = lax.iota(jnp.int32, NLANE)

        def group(g, carry, c=c, lane=lane):
            base_p = g * NLANE
            for p in range(NLANE):
                pair = base_p + p
                acc = (rows_in_v[pair, pl.ds(0, NLANE)]
                       * rows_out_v[pair, pl.ds(0, NLANE)])
                for j in range(1, NVEC):
                    acc = acc + (rows_in_v[pair, pl.ds(j * NLANE, NLANE)]
                                 * rows_out_v[pair, pl.ds(j * NLANE, NLANE)])
                prods_v[p, :] = acc
            dot = plsc.load_gather(
                prods_v, [lane, jnp.zeros((NLANE,), jnp.int32)])
            for j in range(1, NLANE):
                dot = dot + plsc.load_gather(
                    prods_v, [lane, jnp.full((NLANE,), j, jnp.int32)])
            out_v[pl.ds(c * CHUNK + base_p, NLANE)] = dot
            return carry

        lax.fori_loop(0, CHUNK // NLANE, group, 0)

    pltpu.sync_copy(out_v, out_hbm.at[pl.ds(base, BPW)])


@functools.partial(
    pl.kernel,
    out_type=jax.ShapeDtypeStruct((BATCH,), jnp.float32),
    mesh=plsc.VectorSubcoreMesh(core_axis_name="c", subcore_axis_name="s"),
    compiler_params=pltpu.CompilerParams(needs_layout_passes=False),
    scratch_types=[
        pltpu.VMEM((BPW,), jnp.int32),
        pltpu.VMEM((BPW,), jnp.int32),
        pltpu.VMEM((CHUNK, DIM), jnp.float32),
        pltpu.VMEM((CHUNK, DIM), jnp.float32),
        pltpu.VMEM((CHUNK, DIM), jnp.float32),
        pltpu.VMEM((CHUNK, DIM), jnp.float32),
        pltpu.VMEM((NLANE, NLANE), jnp.float32),
        pltpu.VMEM((BPW,), jnp.float32),
        pltpu.SemaphoreType.DMA,
        pltpu.SemaphoreType.DMA,
    ],
)
def _sc_dot(w_in_hbm, w_out_hbm, idx_in_hbm, idx_out_hbm, out_hbm,
            idx_in_v, idx_out_v, rows_in_0, rows_out_0, rows_in_1,
            rows_out_1, prods_v, out_v, sem_0, sem_1):
    _body(w_in_hbm, w_out_hbm, idx_in_hbm, idx_out_hbm, out_hbm,
          idx_in_v, idx_out_v, rows_in_0, rows_out_0, rows_in_1, rows_out_1,
          prods_v, out_v, sem_0, sem_1)


def kernel(x, W_in, W_out):
    idx_in = x[:, 0].astype(jnp.int32)
    idx_out = x[:, 1].astype(jnp.int32)
    return _sc_dot(W_in, W_out, idx_in, idx_out)
